# baseline (speedup 1.0000x reference)
import functools

import jax
import jax.numpy as jnp
from jax import lax
from jax.experimental import pallas as pl
from jax.experimental.pallas import tpu as pltpu
from jax.experimental.pallas import tpu_sc as plsc

_BATCH = 16384
_HIST = 50
_EMBED = 64
_VOCAB2 = 500000
_NC = 2
_NS = 16
_NW = _NC * _NS
_BB = 128
_NBB = _BATCH // _BB
_BB_PER_W = _NBB // _NW
_UNITS = _BB_PER_W * _HIST
_NBUF = 4
_LOOK = 2

_mesh = plsc.VectorSubcoreMesh(core_axis_name="c", subcore_axis_name="s")


@functools.partial(
    pl.kernel,
    mesh=_mesh,
    out_type=jax.ShapeDtypeStruct((_HIST, _EMBED, _BATCH), jnp.float32),
    scratch_types=[
        pltpu.VMEM((_HIST, _BB), jnp.int32),
        pltpu.VMEM((_NBUF * _BB,), jnp.int32),
        pltpu.VMEM((_NBUF * _BB,), jnp.int32),
        pltpu.VMEM((_NBUF, _BB, 2 * _EMBED), jnp.float32),
        pltpu.VMEM((_NBUF, _EMBED, _BB), jnp.float32),
        pltpu.SemaphoreType.DMA((_NBUF,)),
        pltpu.SemaphoreType.DMA((_NBUF,)),
    ],
    compiler_params=pltpu.CompilerParams(needs_layout_passes=False),
)
def _emb_lookup(table_hbm, xt_hbm, out_hbm, idxblk, vrow, prow, rows_v, tr_v,
                gsem, ssem):
    wid = lax.axis_index("s") * _NC + lax.axis_index("c")
    bb0 = wid * _BB_PER_W

    def unit_pos(u):
        bbi = u // _HIST
        return bbi, u - bbi * _HIST

    def fire_gather(u, b):
        bbi, h = unit_pos(u)

        @pl.when(h == 0)
        def _():
            pltpu.sync_copy(
                xt_hbm.at[:, pl.ds((bb0 + bbi) * _BB, _BB)], idxblk
            )

        def cp(k, carry):
            v = idxblk[h, pl.ds(k * 16, 16)]
            vrow[pl.ds(b * _BB + k * 16, 16)] = v
            prow[pl.ds(b * _BB + k * 16, 16)] = v >> 1
            return carry

        lax.fori_loop(0, _BB // 16, cp, 0, unroll=True)
        pltpu.async_copy(
            table_hbm.at[prow.at[pl.ds(b * _BB, _BB)]],
            rows_v.at[b],
            gsem.at[b],
        )

    def wait_gather(b):
        pltpu.make_async_copy(
            table_hbm.at[pl.ds(0, _BB)],
            rows_v.at[b],
            gsem.at[b],
        ).wait()

    def fire_store(u, b):
        bbi, h = unit_pos(u)
        pltpu.async_copy(
            tr_v.at[b],
            out_hbm.at[h, :, pl.ds((bb0 + bbi) * _BB, _BB)],
            ssem.at[b],
        )

    def wait_store(b):
        pltpu.make_async_copy(
            tr_v.at[b],
            out_hbm.at[0, :, pl.ds(0, _BB)],
            ssem.at[b],
        ).wait()

    def transpose(b):
        rows2d = rows_v.at[b]

        def body(g, carry):
            jv = lax.iota(jnp.int32, 16) + g * 16
            av = vrow[pl.ds(b * _BB + g * 16, 16)]
            offv = (av & 1) * _EMBED
            for e in range(_EMBED):
                vals = plsc.load_gather(rows2d, [jv, offv + e])
                tr_v[b, e, pl.ds(g * 16, 16)] = vals
            return carry

        lax.fori_loop(0, _BB // 16, body, 0)

    for u in range(_LOOK):
        fire_gather(u, u % _NBUF)

    def loop_body(g, carry):
        u0 = g * _NBUF
        for b in range(_NBUF):
            u = u0 + b

            @pl.when(u >= _NBUF)
            def _():
                wait_store(b)

            @pl.when(u + _LOOK < _UNITS)
            def _():
                fire_gather(u + _LOOK, (b + _LOOK) % _NBUF)

            wait_gather(b)
            transpose(b)
            fire_store(u, b)
        return carry

    lax.fori_loop(0, _UNITS // _NBUF, loop_body, 0)

    for b in range(_NBUF):
        wait_store(b)


def kernel(x, table):
    tbl2 = table.reshape(_VOCAB2, 2 * _EMBED)
    xt = x.T.astype(jnp.int32)
    out = _emb_lookup(tbl2, xt)
    return out.transpose(2, 0, 1)


# speedup vs baseline: 1.1814x; 1.1814x over previous
import functools

import jax
import jax.numpy as jnp
from jax import lax
from jax.experimental import pallas as pl
from jax.experimental.pallas import tpu as pltpu
from jax.experimental.pallas import tpu_sc as plsc

_BATCH = 16384
_HIST = 50
_EMBED = 64
_VOCAB2 = 500000
_NC = 2
_NS = 16
_NW = _NC * _NS
_BB = 128
_BB_PER_W = (_BATCH // _BB) // _NW
_UNITS = _BB_PER_W * _HIST

_mesh = plsc.VectorSubcoreMesh(core_axis_name="c", subcore_axis_name="s")


@functools.partial(
    pl.kernel,
    mesh=_mesh,
    out_type=jax.ShapeDtypeStruct((_HIST, _EMBED, _BATCH), jnp.float32),
    scratch_types=[
        pltpu.VMEM((_UNITS, _BB), jnp.int32),
        pltpu.VMEM((2, _BB, 2 * _EMBED), jnp.float32),
        pltpu.VMEM((2, _EMBED, _BB), jnp.float32),
        pltpu.VMEM((2, 1, _BB), jnp.int32),
        pltpu.SemaphoreType.DMA((2,)),
        pltpu.SemaphoreType.DMA((2,)),
        pltpu.SemaphoreType.DMA((2,)),
    ],
    compiler_params=pltpu.CompilerParams(needs_layout_passes=False),
)
def _emb_lookup(table_hbm, pidx_hbm, poff_hbm, out_hbm, pidx_v, rows_v, tr_v,
                offv, gsem, osem, ssem):
    wid = lax.axis_index("s") * _NC + lax.axis_index("c")
    bb0 = wid * _BB_PER_W

    pltpu.sync_copy(pidx_hbm.at[wid], pidx_v)

    def fire_gather(u, b):
        pltpu.async_copy(poff_hbm.at[wid, u], offv.at[b], osem.at[b])
        pltpu.async_copy(
            table_hbm.at[pidx_v.at[u]], rows_v.at[b], gsem.at[b]
        )

    def wait_gather(b):
        pltpu.make_async_copy(
            table_hbm.at[pl.ds(0, _BB)], rows_v.at[b], gsem.at[b]
        ).wait()
        pltpu.make_async_copy(
            poff_hbm.at[0, 0], offv.at[b], osem.at[b]
        ).wait()

    def fire_store(u, b):
        bbi = u // _HIST
        h = u - bbi * _HIST
        pltpu.async_copy(
            tr_v.at[b],
            out_hbm.at[h, :, pl.ds((bb0 + bbi) * _BB, _BB)],
            ssem.at[b],
        )

    def wait_store(b):
        pltpu.make_async_copy(
            tr_v.at[b], out_hbm.at[0, :, pl.ds(0, _BB)], ssem.at[b]
        ).wait()

    def transpose(b):
        evs = [lax.iota(jnp.int32, 16) + k * 16 for k in range(_EMBED // 16)]

        def body(g, carry):
            pv = offv[b, 0, pl.ds(g * 16, 16)]
            for r in range(16):
                j = g * 16 + r
                off = pv[r]
                jv = jnp.full((16,), j, dtype=jnp.int32)
                for k in range(_EMBED // 16):
                    plsc.store_scatter(
                        tr_v.at[b], [evs[k], jv],
                        rows_v[b, j, pl.ds(off + k * 16, 16)],
                    )
            return carry

        lax.fori_loop(0, _BB // 16, body, 0)

    fire_gather(0, 0)
    fire_gather(1, 1)

    def loop_body(g, carry):
        u0 = g * 2
        for b in range(2):
            u = u0 + b

            @pl.when(u >= 2)
            def _():
                wait_store(b)

            wait_gather(b)
            transpose(b)
            fire_store(u, b)

            @pl.when(u + 2 < _UNITS)
            def _():
                fire_gather(u + 2, b)
        return carry

    lax.fori_loop(0, _UNITS // 2, loop_body, 0)

    wait_store(0)
    wait_store(1)


def kernel(x, table):
    tbl2 = table.reshape(_VOCAB2, 2 * _EMBED)
    xt = x.T.astype(jnp.int32)
    xu = xt.reshape(_HIST, _NW, _BB_PER_W, _BB).transpose(1, 2, 0, 3)
    pidx = (xu >> 1).reshape(_NW, _UNITS, _BB)
    poff = ((xu & 1) << 6).reshape(_NW, _UNITS, 1, _BB)
    out = _emb_lookup(tbl2, pidx, poff)
    return out.transpose(2, 0, 1)


# speedup vs baseline: 1.4212x; 1.2030x over previous
import functools

import jax
import jax.numpy as jnp
from jax import lax
from jax.experimental import pallas as pl
from jax.experimental.pallas import tpu as pltpu
from jax.experimental.pallas import tpu_sc as plsc

_BATCH = 16384
_HIST = 50
_EMBED = 64
_VOCAB2 = 500000
_NC = 2
_NS = 16
_NW = _NC * _NS
_BB = 128
_BB_PER_W = (_BATCH // _BB) // _NW
_UNITS = _BB_PER_W * _HIST

_mesh = plsc.VectorSubcoreMesh(core_axis_name="c", subcore_axis_name="s")


@functools.partial(
    pl.kernel,
    mesh=_mesh,
    out_type=jax.ShapeDtypeStruct((_HIST, _EMBED, _BATCH), jnp.float32),
    scratch_types=[
        pltpu.VMEM((_UNITS, _BB), jnp.int32),
        pltpu.VMEM((2, _BB, 2 * _EMBED), jnp.float32),
        pltpu.VMEM((2, _EMBED, _BB), jnp.float32),
        pltpu.VMEM((2, 1, _BB), jnp.int32),
        pltpu.SemaphoreType.DMA((2,)),
        pltpu.SemaphoreType.DMA((2,)),
        pltpu.SemaphoreType.DMA((2,)),
    ],
    compiler_params=pltpu.CompilerParams(needs_layout_passes=False),
)
def _emb_lookup(table_hbm, pidx_hbm, poff_hbm, out_hbm, pidx_v, rows_v, tr_v,
                offv, gsem, osem, ssem):
    wid = lax.axis_index("s") * _NC + lax.axis_index("c")
    bb0 = wid * _BB_PER_W

    pltpu.sync_copy(pidx_hbm.at[wid], pidx_v)

    def fire_gather(u, b):
        pltpu.async_copy(poff_hbm.at[wid, u], offv.at[b], osem.at[b])
        pltpu.async_copy(
            table_hbm.at[pidx_v.at[u]], rows_v.at[b], gsem.at[b]
        )

    def wait_gather(b):
        pltpu.make_async_copy(
            table_hbm.at[pl.ds(0, _BB)], rows_v.at[b], gsem.at[b]
        ).wait()
        pltpu.make_async_copy(
            poff_hbm.at[0, 0], offv.at[b], osem.at[b]
        ).wait()

    def fire_store(u, b):
        bbi = u // _HIST
        h = u - bbi * _HIST
        pltpu.async_copy(
            tr_v.at[b],
            out_hbm.at[h, :, pl.ds((bb0 + bbi) * _BB, _BB)],
            ssem.at[b],
        )

    def wait_store(b):
        pltpu.make_async_copy(
            tr_v.at[b], out_hbm.at[0, :, pl.ds(0, _BB)], ssem.at[b]
        ).wait()

    def transpose(b):
        rows2d = rows_v.at[b]
        depth = 8

        def body(g, carry):
            jv = lax.iota(jnp.int32, 16) + g * 16
            ov = offv[b, 0, pl.ds(g * 16, 16)]
            vals = []
            for e in range(_EMBED):
                vals.append(plsc.load_gather(rows2d, [jv, ov + e]))
                if e >= depth:
                    tr_v[b, e - depth, pl.ds(g * 16, 16)] = vals[e - depth]
            for e in range(_EMBED - depth, _EMBED):
                tr_v[b, e, pl.ds(g * 16, 16)] = vals[e]
            return carry

        lax.fori_loop(0, _BB // 16, body, 0)

    fire_gather(0, 0)
    fire_gather(1, 1)

    def loop_body(g, carry):
        u0 = g * 2
        for b in range(2):
            u = u0 + b

            @pl.when(u >= 2)
            def _():
                wait_store(b)

            wait_gather(b)
            transpose(b)
            fire_store(u, b)

            @pl.when(u + 2 < _UNITS)
            def _():
                fire_gather(u + 2, b)
        return carry

    lax.fori_loop(0, _UNITS // 2, loop_body, 0)

    wait_store(0)
    wait_store(1)


def kernel(x, table):
    tbl2 = table.reshape(_VOCAB2, 2 * _EMBED)
    xt = x.T.astype(jnp.int32)
    xu = xt.reshape(_HIST, _NW, _BB_PER_W, _BB).transpose(1, 2, 0, 3)
    pidx = (xu >> 1).reshape(_NW, _UNITS, _BB)
    poff = ((xu & 1) << 6).reshape(_NW, _UNITS, 1, _BB)
    out = _emb_lookup(tbl2, pidx, poff)
    return out.transpose(2, 0, 1)


# speedup vs baseline: 1.4217x; 1.0003x over previous
import functools

import jax
import jax.numpy as jnp
from jax import lax
from jax.experimental import pallas as pl
from jax.experimental.pallas import tpu as pltpu
from jax.experimental.pallas import tpu_sc as plsc

_BATCH = 16384
_HIST = 50
_EMBED = 64
_VOCAB2 = 500000
_NC = 2
_NS = 16
_NW = _NC * _NS
_BB = 128
_BB_PER_W = (_BATCH // _BB) // _NW
_UNITS = _BB_PER_W * _HIST

_mesh = plsc.VectorSubcoreMesh(core_axis_name="c", subcore_axis_name="s")


@functools.partial(
    pl.kernel,
    mesh=_mesh,
    out_type=jax.ShapeDtypeStruct((_HIST, _EMBED, _BATCH), jnp.float32),
    scratch_types=[
        pltpu.VMEM((_UNITS, _BB), jnp.int32),
        pltpu.VMEM((4, _BB, 2 * _EMBED), jnp.float32),
        pltpu.VMEM((2, _EMBED, _BB), jnp.float32),
        pltpu.VMEM((4, 1, _BB), jnp.int32),
        pltpu.SemaphoreType.DMA((4,)),
        pltpu.SemaphoreType.DMA((4,)),
        pltpu.SemaphoreType.DMA((2,)),
    ],
    compiler_params=pltpu.CompilerParams(needs_layout_passes=False),
)
def _emb_lookup(table_hbm, pidx_hbm, poff_hbm, out_hbm, pidx_v, rows_v, tr_v,
                offv, gsem, osem, ssem):
    wid = lax.axis_index("s") * _NC + lax.axis_index("c")
    bb0 = wid * _BB_PER_W

    pltpu.sync_copy(pidx_hbm.at[wid], pidx_v)

    def fire_gather(u, b):
        pltpu.async_copy(poff_hbm.at[wid, u], offv.at[b], osem.at[b])
        pltpu.async_copy(
            table_hbm.at[pidx_v.at[u]], rows_v.at[b], gsem.at[b]
        )

    def wait_gather(b):
        pltpu.make_async_copy(
            table_hbm.at[pl.ds(0, _BB)], rows_v.at[b], gsem.at[b]
        ).wait()
        pltpu.make_async_copy(
            poff_hbm.at[0, 0], offv.at[b], osem.at[b]
        ).wait()

    def fire_store(u, b):
        bbi = u // _HIST
        h = u - bbi * _HIST
        pltpu.async_copy(
            tr_v.at[b],
            out_hbm.at[h, :, pl.ds((bb0 + bbi) * _BB, _BB)],
            ssem.at[b],
        )

    def wait_store(b):
        pltpu.make_async_copy(
            tr_v.at[b], out_hbm.at[0, :, pl.ds(0, _BB)], ssem.at[b]
        ).wait()

    def transpose(b, bt):
        rows2d = rows_v.at[b]
        depth = 8

        def body(g, carry):
            jv = lax.iota(jnp.int32, 16) + g * 16
            ov = offv[b, 0, pl.ds(g * 16, 16)]
            vals = []
            for e in range(_EMBED):
                vals.append(plsc.load_gather(rows2d, [jv, ov + e]))
                if e >= depth:
                    tr_v[bt, e - depth, pl.ds(g * 16, 16)] = vals[e - depth]
            for e in range(_EMBED - depth, _EMBED):
                tr_v[bt, e, pl.ds(g * 16, 16)] = vals[e]
            return carry

        lax.fori_loop(0, _BB // 16, body, 0)

    fire_gather(0, 0)
    fire_gather(1, 1)

    def loop_body(g, carry):
        u0 = g * 4
        for i in range(4):
            u = u0 + i
            br = i
            bt = i % 2

            @pl.when(u >= 2)
            def _():
                wait_store(bt)

            @pl.when(u + 2 < _UNITS)
            def _():
                fire_gather(u + 2, (i + 2) % 4)

            wait_gather(br)
            transpose(br, bt)
            fire_store(u, bt)
        return carry

    lax.fori_loop(0, _UNITS // 4, loop_body, 0)

    wait_store(0)
    wait_store(1)


def kernel(x, table):
    tbl2 = table.reshape(_VOCAB2, 2 * _EMBED)
    xt = x.T.astype(jnp.int32)
    xu = xt.reshape(_HIST, _NW, _BB_PER_W, _BB).transpose(1, 2, 0, 3)
    pidx = (xu >> 1).reshape(_NW, _UNITS, _BB)
    poff = ((xu & 1) << 6).reshape(_NW, _UNITS, 1, _BB)
    out = _emb_lookup(tbl2, pidx, poff)
    return out.transpose(2, 0, 1)


# speedup vs baseline: 1.4712x; 1.0348x over previous
import functools

import jax
import jax.numpy as jnp
from jax import lax
from jax.experimental import pallas as pl
from jax.experimental.pallas import tpu as pltpu
from jax.experimental.pallas import tpu_sc as plsc

_BATCH = 16384
_HIST = 50
_EMBED = 64
_N = _BATCH * _HIST
_NC = 2
_NS = 16
_NW = _NC * _NS
_PER_W = _N // _NW
_CHUNK = 128
_NCH = _PER_W // _CHUNK
_NBUF = 8
_LOOK = 4
_NGRP = _NCH // _NBUF

_mesh = plsc.VectorSubcoreMesh(core_axis_name="c", subcore_axis_name="s")


@functools.partial(
    pl.kernel,
    mesh=_mesh,
    out_type=jax.ShapeDtypeStruct((_N, _EMBED), jnp.float32),
    scratch_types=[
        pltpu.VMEM((_NCH, _CHUNK), jnp.int32),
        pltpu.VMEM((_NBUF, _CHUNK, _EMBED), jnp.float32),
        pltpu.SemaphoreType.DMA((_NBUF,)),
        pltpu.SemaphoreType.DMA((_NBUF,)),
    ],
    compiler_params=pltpu.CompilerParams(use_tc_tiling_on_sc=False),
)
def _emb_lookup(table_hbm, idx_hbm, out_hbm, idx_v, rows_v, gsem, ssem):
    wid = lax.axis_index("s") * _NC + lax.axis_index("c")
    base = wid * _PER_W

    pltpu.sync_copy(idx_hbm.at[pl.ds(wid * _NCH, _NCH)], idx_v)

    def start_gather(chunk, b):
        pltpu.async_copy(table_hbm.at[idx_v.at[chunk]], rows_v.at[b], gsem.at[b])

    def wait_gather(b):
        pltpu.make_async_copy(
            out_hbm.at[pl.ds(0, _CHUNK)], rows_v.at[b], gsem.at[b]
        ).wait()

    def start_store(chunk, b):
        pltpu.async_copy(
            rows_v.at[b], out_hbm.at[pl.ds(base + chunk * _CHUNK, _CHUNK)], ssem.at[b]
        )

    def wait_store(b):
        pltpu.make_async_copy(
            rows_v.at[b], out_hbm.at[pl.ds(0, _CHUNK)], ssem.at[b]
        ).wait()

    for b in range(_LOOK):
        start_gather(b, b)

    def loop_body(g, carry):
        c0 = g * _NBUF
        for b in range(_NBUF):
            c = c0 + b
            cn = c + _LOOK
            bn = (b + _LOOK) % _NBUF

            @pl.when(cn >= _NBUF)
            def _():
                wait_store(bn)

            @pl.when(cn < _NCH)
            def _():
                start_gather(cn, bn)

            wait_gather(b)
            start_store(c, b)
        return carry

    lax.fori_loop(0, _NGRP, loop_body, 0)

    for i in range(_LOOK):
        wait_store((_LOOK + i) % _NBUF)


def kernel(x, table):
    idx = x.reshape(-1).astype(jnp.int32).reshape(_N // _CHUNK, _CHUNK)
    out = _emb_lookup(table, idx)
    return out.reshape(_BATCH, _HIST, _EMBED)
